# baseline (speedup 1.0000x reference)
import functools

import jax
import jax.numpy as jnp
from jax import lax
from jax.experimental import pallas as pl
from jax.experimental.pallas import tpu as pltpu
from jax.experimental.pallas import tpu_sc as plsc

B, L, D = 16, 4096, 1024
NC, NS = 2, 16
NW = NC * NS
SLAB = L // NW
CHRJ = 2
NBUF = 3
CHUNK = CHRJ * B
NCHUNK = SLAB // CHRJ


def _sc_body(x_hbm, tok_hbm, params_hbm, out_hbm,
             vbuf, pbuf, prow_v, idx2_v, sem_in, sem_out, sem_m):
    wid = lax.axis_index("s") * NC + lax.axis_index("c")
    j0 = wid * SLAB

    pltpu.sync_copy(
        params_hbm.at[pl.ds(pl.multiple_of(wid * 8, 8), 8)], prow_v)
    lens = prow_v[0]

    lane = lax.broadcasted_iota(jnp.int32, (NS,), 0)
    in_lane = lane * L

    def set_idx(cur, jrow):
        idx2_v[cur, 0:NS] = in_lane + jrow
        idx2_v[cur, NS:2 * NS] = in_lane + (jrow + 1)

    loads = [None, None, None]
    stores = [None, None, None]

    def start_load(c):
        buf = c % 3
        set_idx(buf, j0 + c * CHRJ)
        loads[buf] = pltpu.async_copy(
            x_hbm.at[idx2_v.at[buf]],
            vbuf.at[pl.ds(buf * CHUNK, CHUNK)], sem_in.at[buf])

    start_load(0)
    start_load(1)
    for i in range(NCHUNK):
        cur = i % 3
        if i + 2 < NCHUNK:
            if i >= 1:
                stores[(i - 1) % 3].wait()
            start_load(i + 2)
        loads[cur].wait()
        stores[cur] = pltpu.async_copy(
            vbuf.at[pl.ds(cur * CHUNK, CHUNK)],
            out_hbm.at[pl.ds(pl.multiple_of((j0 + i * CHRJ) * B, 8), CHUNK)],
            sem_out.at[cur])
    for buf in range(3):
        stores[buf].wait()

    for b in range(B):
        last_b = lens[b]

        @pl.when((last_b >= j0) & (last_b < j0 + SLAB))
        def _patch(last_b=last_b, b=b):
            blk = out_hbm.at[pl.ds(pl.multiple_of(last_b * B, 8), B)]
            pltpu.async_copy(blk, pbuf, sem_m).wait()
            pltpu.sync_copy(tok_hbm, pbuf.at[pl.ds(b, 1)])
            pltpu.async_copy(pbuf, blk, sem_m).wait()

    @pl.when(wid == NW - 1)
    def _wrap():
        for b in range(B):
            pltpu.sync_copy(x_hbm.at[pl.ds(b * L, 1)],
                            pbuf.at[pl.ds(b, 1)])
        pltpu.async_copy(pbuf, out_hbm.at[pl.ds(L * B, B)], sem_m).wait()


def _ts_body(lens_ref, ts_ref, out_ts_ref, out_len_ref):
    cols = lax.broadcasted_iota(jnp.int32, (1, L), 1)
    for b in range(B):
        last = lens_ref[b]
        last_m1 = jnp.maximum(last - 1, 0)
        row = ts_ref[b:b + 1, :]
        last_ts = jnp.sum(jnp.where(cols == last_m1, row, 0.0))
        out_ts_ref[b:b + 1, :L] = jnp.where(cols == last, last_ts, row)
        out_ts_ref[b:b + 1, L:L + 1] = row[:, 0:1]
        out_len_ref[b] = last + 1


def kernel(x_payload, timestamps, seq_lens, token):
    seq_lens = seq_lens.astype(jnp.int32)
    token2 = token.reshape(1, D)

    params = jnp.zeros((NW * 8, 16), jnp.int32)
    params = params.at[jnp.arange(NW) * 8].set(
        jnp.broadcast_to(seq_lens, (NW, B)))

    x_flat = x_payload.reshape(B * L, D)

    mesh = plsc.VectorSubcoreMesh(core_axis_name="c", subcore_axis_name="s")
    sc_copy = functools.partial(
        pl.kernel,
        out_type=jax.ShapeDtypeStruct(((L + 1) * B, D), x_payload.dtype),
        mesh=mesh,
        scratch_types=[
            pltpu.VMEM((3 * CHUNK, D), jnp.float32),
            pltpu.VMEM((B, D), jnp.float32),
            pltpu.VMEM((8, 16), jnp.int32),
            pltpu.VMEM((3, CHUNK), jnp.int32),
            pltpu.SemaphoreType.DMA((3,)),
            pltpu.SemaphoreType.DMA((3,)),
            pltpu.SemaphoreType.DMA,
        ],
    )(_sc_body)
    out_flat = sc_copy(x_flat, token2, params)
    new_x = out_flat.reshape(L + 1, B, D).transpose(1, 0, 2)

    new_ts, new_len = pl.pallas_call(
        _ts_body,
        in_specs=[
            pl.BlockSpec(memory_space=pltpu.SMEM),
            pl.BlockSpec(memory_space=pltpu.VMEM),
        ],
        out_specs=[
            pl.BlockSpec(memory_space=pltpu.VMEM),
            pl.BlockSpec(memory_space=pltpu.SMEM),
        ],
        out_shape=[
            jax.ShapeDtypeStruct((B, L + 1), timestamps.dtype),
            jax.ShapeDtypeStruct((B,), jnp.int32),
        ],
    )(seq_lens, timestamps)
    return new_x, new_len, new_ts, new_len
